# baseline (speedup 1.0000x reference)
import functools

import jax
import jax.numpy as jnp
from jax import lax
from jax.experimental import pallas as pl
from jax.experimental.pallas import tpu as pltpu
from jax.experimental.pallas import tpu_sc as plsc

G = 16
H = 32
NUM_LAYERS = 4
NUM_ITER = 30
ELASTICITY = 0.01
_BNS = 1.0 / (1.0 + 1e-5) ** 0.5

NP = 100352
PR = 16
RB = 2048
NB = NP // RB


def _col(v):
    return v.reshape(NP, 1)


def _plane(v):
    return v.reshape(NP // 128, 128)


def _layer1_body(x_ref, ax0_ref, ax1_ref, m0_ref, p0_ref, p1_ref, w1_ref,
                 b1_ref, w2_ref, b2_ref, gb_ref, h0_ref, h1_ref, m1_ref):
    m1 = jnp.where(m0_ref[...] + (p0_ref[...] + p1_ref[...]) > 0, 1.0, 0.0)
    m1_ref[...] = m1
    h = x_ref[...] + (ax0_ref[...] + ax1_ref[...])
    y = h * w1_ref[...] + b1_ref[...]
    y = jnp.maximum(y, 0.0)
    y = jnp.maximum(y @ w2_ref[...] + b2_ref[...], 0.0)
    y = gb_ref[0:1, :] * y + gb_ref[1:2, :]
    y = y * m1
    h0_ref[...] = y[:, :16]
    h1_ref[...] = y[:, 16:]


def _gin_body(h0_ref, h1_ref, a0_ref, a1_ref, mp_ref, p0_ref, p1_ref,
              w1_ref, w2_ref, c_ref, o0_ref, o1_ref, mn_ref):
    mnew = jnp.where(mp_ref[...] + (p0_ref[...] + p1_ref[...]) > 0, 1.0, 0.0)
    mn_ref[...] = mnew
    h = jnp.concatenate([h0_ref[...], h1_ref[...]], axis=1)
    y = jnp.concatenate([h0_ref[...] + a0_ref[...],
                         h1_ref[...] + a1_ref[...]], axis=1)
    y = jnp.maximum(y @ w1_ref[...] + c_ref[0:1, :], 0.0)
    y = jnp.maximum(y @ w2_ref[...] + c_ref[1:2, :], 0.0)
    y = c_ref[2:3, :] * y + c_ref[3:4, :]
    hn = (h + y) * mnew
    hn = c_ref[4:5, :] * hn + c_ref[5:6, :]
    o0_ref[...] = hn[:, :16]
    o1_ref[...] = hn[:, 16:]


def _head_body(h0_ref, h1_ref, m_ref, w1_ref, c_ref, w2_ref, z_ref):
    y = jnp.concatenate([h0_ref[...], h1_ref[...]], axis=1)
    y = y @ w1_ref[...] + c_ref[0:1, :]
    y = jnp.where(y >= 0, y, 0.01 * y)
    y = y * m_ref[...]
    y = c_ref[1:2, :] * y + c_ref[2:3, :]
    z = y @ w2_ref[...] + c_ref[3:4, 0:1]
    z = jnp.where(z >= 0, z, 0.01 * z)
    z_ref[...] = z * m_ref[...]


def _row16(scalars):
    lidx = jax.lax.broadcasted_iota(jnp.int32, (1, G), 1)
    acc = jnp.zeros((1, G), jnp.float32)
    for g in range(G):
        acc = acc + jnp.where(lidx == g, scalars[g], 0.0)
    return acc


def _solver_body(z_ref, m_ref, x_ref, d0_ref, d1_ref, b_ref,
                 fe_ref, tr_ref, p_ref):
    b = b_ref[...]
    beq = [b == g for g in range(G)]
    zero = jnp.zeros_like(z_ref[...])

    def bcast(vals):
        acc = zero
        for g in range(G):
            acc = acc + jnp.where(beq[g], vals[g], 0.0)
        return acc

    def ssum(v):
        return [jnp.sum(jnp.where(beq[g], v, 0.0)) for g in range(G)]

    m = m_ref[...]
    deg = d0_ref[...] + d1_ref[...]
    z = z_ref[...]
    bmaxn = bcast([jnp.max(jnp.where(beq[g], z, -jnp.inf))
                   for g in range(G)])
    bminn = bcast([jnp.min(jnp.where(beq[g], z, jnp.inf))
                   for g in range(G)])
    h = (z - bminn) / (bmaxn + 1e-6 - bminn)
    h = h * m + m * 1e-6
    xfilt = h + x_ref[...]

    tv = ssum(deg)
    target = [(tr_ref[0, g] * fe_ref[0, g] * 0.85 + 0.1) * (tv[g] + 1e-6)
              for g in range(G)]
    dm = deg * m
    xdm = xfilt * dm
    sum_dm = ssum(dm)

    def body(it, carry):
        a, done = carry
        a_b = bcast(a)
        keep = a_b * xfilt < 1
        dmk = jnp.where(keep, dm, 0.0)
        xdmk = jnp.where(keep, xdm, 0.0)
        s1 = ssum(dmk)
        s2 = ssum(xdmk)
        a_new = [jnp.where(done, a[g],
                           (target[g] - (sum_dm[g] - s1[g])) / (s2[g] + 1e-5))
                 for g in range(G)]
        a_nb = bcast(a_new)
        vc = jnp.clip(a_nb * xfilt, 0.0, 1.0) * deg
        vcs = ssum(vc)
        checks = [(jnp.abs(vcs[g] - target[g]) <= ELASTICITY * target[g])
                  .astype(jnp.float32) for g in range(G)]
        csum = functools.reduce(lambda p, q: p + q, checks)
        done = jnp.logical_or(done, csum >= G)
        return tuple(a_new), done

    a0 = tuple(jnp.float32(1.0) for _ in range(G))
    a, _ = jax.lax.fori_loop(0, NUM_ITER, body, (a0, jnp.array(False)))
    p_ref[...] = jnp.clip(bcast(list(a)) * xfilt * m, 0.0, 1.0)


def _loss_body(p_ref, d0_ref, d1_ref, s0_ref, s1_ref, b_ref, e_ref, l_ref):
    @pl.when(pl.program_id(0) == 0)
    def _():
        e_ref[...] = jnp.zeros((1, G), jnp.float32)
    v = p_ref[...] * ((d0_ref[...] + d1_ref[...])
                      - (s0_ref[...] + s1_ref[...]))
    b = b_ref[...]
    e_ref[...] += _row16([jnp.sum(jnp.where(b == g, v, 0.0))
                          for g in range(G)])

    @pl.when(pl.program_id(0) == pl.num_programs(0) - 1)
    def _():
        l_ref[...] = jnp.full((1, 1), 1.0, jnp.float32) * (
            jnp.sum(e_ref[...]) / G)


def _cspec(r, c):
    return pl.BlockSpec((r, c), lambda i: (i, 0))


def _kspec(shape):
    return pl.BlockSpec(shape, lambda i: (0, 0))


def _f32(shape):
    return jax.ShapeDtypeStruct(shape, jnp.float32)


NS = 16
NW = 2 * NS
CS = 10000
CV = 1000
SR = NP // NS
VR = NP // NS
ZR = 98
NZC = VR // ZR


def _mesh():
    return plsc.VectorSubcoreMesh(core_axis_name="c", subcore_axis_name="s")


_SC_PARAMS = pltpu.CompilerParams(use_tc_tiling_on_sc=False)


def _zero_vec(zb, nelem):
    def zf(j, _):
        zb[pl.ds(j * 16, 16)] = jnp.zeros((16,), jnp.float32)
        return 0
    lax.fori_loop(0, nelem // 16, zf, 0)


def _body_psum(e, vals, srci, dsti, out, acc, idx_v, val_v, zb, sem):
    c = lax.axis_index("c")
    s = lax.axis_index("s")
    _zero_vec(zb, SR)
    pltpu.sync_copy(zb, acc.at[pl.ds(s * SR, SR)])
    plsc.subcore_barrier()
    ew = e // NW
    t0 = (c * NS + s) * ew

    def ch(i, _):
        off = t0 + i * CS
        pltpu.sync_copy(srci.at[pl.ds(off, CS)], idx_v)
        pltpu.async_copy(vals.at[idx_v], val_v, sem).wait()
        pltpu.sync_copy(dsti.at[pl.ds(off, CS)], idx_v)
        pltpu.sync_copy(val_v, acc.at[idx_v], add=True)
        return 0
    lax.fori_loop(0, ew // CS, ch, 0)
    plsc.subcore_barrier()
    pltpu.sync_copy(acc.at[pl.ds(s * SR, SR)], zb)
    pltpu.sync_copy(zb, out.at[pl.ds(c * NP + s * SR, SR)])


def _body_pxdeg(e, vals, srci, dsti, outx, outd, accx, accd, idx_v, val_v,
                ones_v, zb, sem):
    c = lax.axis_index("c")
    s = lax.axis_index("s")
    _zero_vec(zb, SR)
    pltpu.sync_copy(zb, accx.at[pl.ds(s * SR, SR)])
    pltpu.sync_copy(zb, accd.at[pl.ds(s * SR, SR)])

    def of(j, _):
        ones_v[pl.ds(j * 16, 16)] = jnp.ones((16,), jnp.float32)
        return 0
    lax.fori_loop(0, CS // 16, of, 0)
    plsc.subcore_barrier()

    ew = e // NW
    t0 = (c * NS + s) * ew

    def ch(i, _):
        off = t0 + i * CS
        pltpu.sync_copy(srci.at[pl.ds(off, CS)], idx_v)
        pltpu.async_copy(vals.at[idx_v], val_v, sem).wait()
        pltpu.sync_copy(ones_v, accd.at[idx_v], add=True)
        pltpu.sync_copy(dsti.at[pl.ds(off, CS)], idx_v)
        pltpu.sync_copy(val_v, accx.at[idx_v], add=True)
        return 0
    lax.fori_loop(0, ew // CS, ch, 0)
    plsc.subcore_barrier()
    pltpu.sync_copy(accx.at[pl.ds(s * SR, SR)], zb)
    pltpu.sync_copy(zb, outx.at[pl.ds(c * NP + s * SR, SR)])
    pltpu.sync_copy(accd.at[pl.ds(s * SR, SR)], zb)
    pltpu.sync_copy(zb, outd.at[pl.ds(c * NP + s * SR, SR)])


def _body_gin(e, hcat, srci, dsti, outcat, acc, idx_v, didx_v, val_v, zb2,
              sem):
    c = lax.axis_index("c")
    s = lax.axis_index("s")

    def zf(j, _):
        zb2[j, :] = jnp.zeros((16,), jnp.float32)
        return 0
    lax.fori_loop(0, ZR, zf, 0)

    def zc(k, _):
        pltpu.sync_copy(zb2, acc.at[pl.ds(s * VR + k * ZR, ZR)])
        return 0
    lax.fori_loop(0, NZC, zc, 0)
    plsc.subcore_barrier()

    et = e // NS
    t0 = s * et

    def ch(i, _):
        off = t0 + i * CV
        pltpu.sync_copy(srci.at[pl.ds(off, CV)], idx_v)

        def rb(j, _):
            sl = pl.ds(j * 16, 16)
            idx_v[sl] = idx_v[sl] + c * NP
            return 0
        lax.fori_loop(0, CV // 16, rb, 0)
        pltpu.async_copy(hcat.at[idx_v], val_v, sem).wait()
        pltpu.sync_copy(dsti.at[pl.ds(off, CV)], didx_v)
        pltpu.sync_copy(val_v, acc.at[didx_v], add=True)
        return 0
    lax.fori_loop(0, et // CV, ch, 0)
    plsc.subcore_barrier()

    def oc(k, _):
        pltpu.sync_copy(acc.at[pl.ds(s * VR + k * ZR, ZR)], zb2)
        pltpu.sync_copy(zb2, outcat.at[pl.ds(c * NP + s * VR + k * ZR, ZR)])
        return 0
    lax.fori_loop(0, NZC, oc, 0)


def _sc_psum(vals, srci, dsti):
    e = srci.shape[0]
    return pl.kernel(
        functools.partial(_body_psum, e),
        out_type=jax.ShapeDtypeStruct((2 * NP,), jnp.float32),
        mesh=_mesh(),
        scratch_types=[
            pltpu.VMEM_SHARED((NP,), jnp.float32),
            pltpu.VMEM((CS,), jnp.int32),
            pltpu.VMEM((CS,), jnp.float32),
            pltpu.VMEM((SR,), jnp.float32),
            pltpu.SemaphoreType.DMA,
        ],
        compiler_params=_SC_PARAMS,
    )(vals, srci, dsti)


def _sc_pxdeg(x, srci, dsti):
    e = srci.shape[0]
    return pl.kernel(
        functools.partial(_body_pxdeg, e),
        out_type=[jax.ShapeDtypeStruct((2 * NP,), jnp.float32),
                  jax.ShapeDtypeStruct((2 * NP,), jnp.float32)],
        mesh=_mesh(),
        scratch_types=[
            pltpu.VMEM_SHARED((NP,), jnp.float32),
            pltpu.VMEM_SHARED((NP,), jnp.float32),
            pltpu.VMEM((CS,), jnp.int32),
            pltpu.VMEM((CS,), jnp.float32),
            pltpu.VMEM((CS,), jnp.float32),
            pltpu.VMEM((SR,), jnp.float32),
            pltpu.SemaphoreType.DMA,
        ],
        compiler_params=_SC_PARAMS,
    )(x, srci, dsti)


def _sc_edge_gin(hcat, srci, dsti):
    e = srci.shape[0]
    return pl.kernel(
        functools.partial(_body_gin, e),
        out_type=jax.ShapeDtypeStruct((2 * NP, 16), jnp.float32),
        mesh=_mesh(),
        scratch_types=[
            pltpu.VMEM_SHARED((NP, 16), jnp.float32),
            pltpu.VMEM((CV,), jnp.int32),
            pltpu.VMEM((CV,), jnp.int32),
            pltpu.VMEM((CV, 16), jnp.float32),
            pltpu.VMEM((ZR, 16), jnp.float32),
            pltpu.SemaphoreType.DMA,
        ],
        compiler_params=_SC_PARAMS,
    )(hcat, srci, dsti)


def kernel(x, edge_index, batch, recfield_vol, total_vol, params):
    n = x.shape[0]
    row, col = edge_index[0], edge_index[1]
    pad = NP - n

    xp = jnp.pad(x, (0, pad))
    batchp = jnp.pad(batch, (0, pad), constant_values=G)

    m0 = jnp.pad((jnp.abs(x) > 0).astype(jnp.float32), (0, pad))
    pm = _sc_psum(m0, row, col).reshape(2, NP)
    px, pd = _sc_pxdeg(xp, row, col)
    pxr = px.reshape(2, NP)

    w1r = params['c1_W1'].reshape(1, H)
    gb = jnp.stack([params['c1_g'] * _BNS, params['c1_bt']])
    h0, h1, mcol = pl.pallas_call(
        _layer1_body,
        grid=(NB,),
        in_specs=[_cspec(RB, 1), _cspec(RB, 1), _cspec(RB, 1), _cspec(RB, 1),
                  _cspec(RB, 1), _cspec(RB, 1),
                  _kspec((1, H)), _kspec((1, H)), _kspec((H, H)),
                  _kspec((1, H)), _kspec((2, H))],
        out_specs=[_cspec(RB, 16), _cspec(RB, 16), _cspec(RB, 1)],
        out_shape=[_f32((NP, 16)), _f32((NP, 16)), _f32((NP, 1))],
    )(_col(xp), _col(pxr[0]), _col(pxr[1]), _col(m0), _col(pm[0]),
      _col(pm[1]), w1r, params['c1_b1'].reshape(1, H), params['c1_W2'],
      params['c1_b2'].reshape(1, H), gb)

    for i in range(NUM_LAYERS - 1):
        pm = _sc_psum(mcol.reshape(NP), row, col).reshape(2, NP)
        acat = _sc_edge_gin(jnp.concatenate([h0, h1], axis=0), row, col)
        a0 = acat[:NP]
        a1 = acat[NP:]
        c = jnp.stack([params['cv%d_b1' % i], params['cv%d_b2' % i],
                       params['cv%d_g' % i] * _BNS, params['cv%d_bt' % i],
                       params['bn%d_g' % i] * _BNS, params['bn%d_bt' % i]])
        h0, h1, mcol = pl.pallas_call(
            _gin_body,
            grid=(NB,),
            in_specs=[_cspec(RB, 16), _cspec(RB, 16), _cspec(RB, 16),
                      _cspec(RB, 16), _cspec(RB, 1), _cspec(RB, 1),
                      _cspec(RB, 1),
                      _kspec((H, H)), _kspec((H, H)), _kspec((6, H))],
            out_specs=[_cspec(RB, 16), _cspec(RB, 16), _cspec(RB, 1)],
            out_shape=[_f32((NP, 16)), _f32((NP, 16)), _f32((NP, 1))],
        )(h0, h1, a0, a1, mcol, _col(pm[0]), _col(pm[1]),
          params['cv%d_W1' % i], params['cv%d_W2' % i], c)

    m4 = mcol.reshape(NP)

    c = jnp.zeros((4, H), jnp.float32)
    c = c.at[0].set(params['lin1_b'])
    c = c.at[1].set(params['bn2_g'] * _BNS)
    c = c.at[2].set(params['bn2_bt'])
    c = c.at[3, 0].set(params['lin2_b'][0])
    z = pl.pallas_call(
        _head_body,
        grid=(NB,),
        in_specs=[_cspec(RB, 16), _cspec(RB, 16), _cspec(RB, 1),
                  _kspec((H, H)), _kspec((4, H)), _kspec((H, 1))],
        out_specs=_cspec(RB, 1),
        out_shape=_f32((NP, 1)),
    )(h0, h1, _col(m4), params['lin1_W'], c, params['lin2_W'])
    z = z.reshape(NP)

    kt = jax.random.fold_in(jax.random.key(1), 7)
    trand = jax.random.uniform(kt, (G,), jnp.float32).reshape(1, G)
    feas = (recfield_vol / total_vol).reshape(1, G)
    pdr = pd.reshape(2, NP // 128, 128)
    full = pl.BlockSpec((NP // 128, 128), lambda: (0, 0))
    probs = pl.pallas_call(
        _solver_body,
        grid=(),
        in_specs=[full, full, full, full, full, full,
                  pl.BlockSpec((1, G), lambda: (0, 0)),
                  pl.BlockSpec((1, G), lambda: (0, 0))],
        out_specs=full,
        out_shape=_f32((NP // 128, 128)),
    )(_plane(z), _plane(m4), _plane(xp), pdr[0], pdr[1], _plane(batchp),
      feas, trand)
    probs = probs.reshape(NP)

    ps = _sc_psum(probs, col, row).reshape(2, NP // 128, 128)
    e, loss = pl.pallas_call(
        _loss_body,
        grid=(NB,),
        in_specs=[_cspec(PR, 128), _cspec(PR, 128), _cspec(PR, 128),
                  _cspec(PR, 128), _cspec(PR, 128), _cspec(PR, 128)],
        out_specs=[_kspec((1, G)), _kspec((1, 1))],
        out_shape=[_f32((1, G)), _f32((1, 1))],
    )(_plane(probs), pdr[0], pdr[1], ps[0], ps[1], _plane(batchp))

    return probs[:n], loss[0, 0]


# speedup vs baseline: 1.0706x; 1.0706x over previous
import functools

import jax
import jax.numpy as jnp
from jax import lax
from jax.experimental import pallas as pl
from jax.experimental.pallas import tpu as pltpu
from jax.experimental.pallas import tpu_sc as plsc

G = 16
H = 32
NUM_LAYERS = 4
NUM_ITER = 30
ELASTICITY = 0.01
_BNS = 1.0 / (1.0 + 1e-5) ** 0.5

NP = 100352
PR = 16
RB = 2048
NB = NP // RB


def _col(v):
    return v.reshape(NP, 1)


def _plane(v):
    return v.reshape(NP // 128, 128)


def _layer1_body(x_ref, ax0_ref, ax1_ref, m_ref, w1_ref, b1_ref, w2_ref,
                 b2_ref, gb_ref, h0_ref, h1_ref):
    h = x_ref[...] + (ax0_ref[...] + ax1_ref[...])
    y = h * w1_ref[...] + b1_ref[...]
    y = jnp.maximum(y, 0.0)
    y = jnp.maximum(y @ w2_ref[...] + b2_ref[...], 0.0)
    y = gb_ref[0:1, :] * y + gb_ref[1:2, :]
    y = y * m_ref[...]
    h0_ref[...] = y[:, :16]
    h1_ref[...] = y[:, 16:]


def _gin_body(h0_ref, h1_ref, a0_ref, a1_ref, m_ref, w1_ref, w2_ref,
              c_ref, o0_ref, o1_ref):
    h = jnp.concatenate([h0_ref[...], h1_ref[...]], axis=1)
    y = jnp.concatenate([h0_ref[...] + a0_ref[...],
                         h1_ref[...] + a1_ref[...]], axis=1)
    y = jnp.maximum(y @ w1_ref[...] + c_ref[0:1, :], 0.0)
    y = jnp.maximum(y @ w2_ref[...] + c_ref[1:2, :], 0.0)
    y = c_ref[2:3, :] * y + c_ref[3:4, :]
    hn = (h + y) * m_ref[...]
    hn = c_ref[4:5, :] * hn + c_ref[5:6, :]
    o0_ref[...] = hn[:, :16]
    o1_ref[...] = hn[:, 16:]


def _head_body(h0_ref, h1_ref, m_ref, w1_ref, c_ref, w2_ref, z_ref):
    y = jnp.concatenate([h0_ref[...], h1_ref[...]], axis=1)
    y = y @ w1_ref[...] + c_ref[0:1, :]
    y = jnp.where(y >= 0, y, 0.01 * y)
    y = y * m_ref[...]
    y = c_ref[1:2, :] * y + c_ref[2:3, :]
    z = y @ w2_ref[...] + c_ref[3:4, 0:1]
    z = jnp.where(z >= 0, z, 0.01 * z)
    z_ref[...] = z * m_ref[...]


def _thr_body(m_ref, p0_ref, p1_ref, o_ref):
    o_ref[...] = jnp.where(m_ref[...] + (p0_ref[...] + p1_ref[...]) > 0,
                           1.0, 0.0)


def _row16(scalars):
    lidx = jax.lax.broadcasted_iota(jnp.int32, (1, G), 1)
    acc = jnp.zeros((1, G), jnp.float32)
    for g in range(G):
        acc = acc + jnp.where(lidx == g, scalars[g], 0.0)
    return acc


def _minmax_body(z_ref, b_ref, mx_ref, mn_ref):
    @pl.when(pl.program_id(0) == 0)
    def _():
        mx_ref[...] = jnp.full((1, G), -jnp.inf, jnp.float32)
        mn_ref[...] = jnp.full((1, G), jnp.inf, jnp.float32)
    z = z_ref[...]
    b = b_ref[...]
    mxs = [jnp.max(jnp.where(b == g, z, -jnp.inf)) for g in range(G)]
    mns = [jnp.min(jnp.where(b == g, z, jnp.inf)) for g in range(G)]
    lidx = jax.lax.broadcasted_iota(jnp.int32, (1, G), 1)
    mxrow = jnp.full((1, G), -jnp.inf, jnp.float32)
    mnrow = jnp.full((1, G), jnp.inf, jnp.float32)
    for g in range(G):
        mxrow = jnp.where(lidx == g, mxs[g], mxrow)
        mnrow = jnp.where(lidx == g, mns[g], mnrow)
    mx_ref[...] = jnp.maximum(mx_ref[...], mxrow)
    mn_ref[...] = jnp.minimum(mn_ref[...], mnrow)


def _solver_body(z_ref, m_ref, x_ref, d0_ref, d1_ref, b_ref, mx_ref, mn_ref,
                 fe_ref, tr_ref, p_ref):
    b = b_ref[...]
    beq = [b == g for g in range(G)]
    zero = jnp.zeros_like(z_ref[...])

    def bcast(vals):
        acc = zero
        for g in range(G):
            acc = acc + jnp.where(beq[g], vals[g], 0.0)
        return acc

    def ssum(v):
        return [jnp.sum(jnp.where(beq[g], v, 0.0)) for g in range(G)]

    m = m_ref[...]
    deg = d0_ref[...] + d1_ref[...]
    bmaxn = bcast([mx_ref[0, g] for g in range(G)])
    bminn = bcast([mn_ref[0, g] for g in range(G)])
    h = (z_ref[...] - bminn) / (bmaxn + 1e-6 - bminn)
    h = h * m + m * 1e-6
    xfilt = h + x_ref[...]

    tv = ssum(deg)
    target = [(tr_ref[0, g] * fe_ref[0, g] * 0.85 + 0.1) * (tv[g] + 1e-6)
              for g in range(G)]
    dm = deg * m
    xdm = xfilt * dm
    sum_dm = ssum(dm)

    def body(it, carry):
        a, done = carry
        a_b = bcast(a)
        keep = a_b * xfilt < 1
        dmk = jnp.where(keep, dm, 0.0)
        xdmk = jnp.where(keep, xdm, 0.0)
        s1 = ssum(dmk)
        s2 = ssum(xdmk)
        a_new = [jnp.where(done, a[g],
                           (target[g] - (sum_dm[g] - s1[g])) / (s2[g] + 1e-5))
                 for g in range(G)]
        a_nb = bcast(a_new)
        vc = jnp.clip(a_nb * xfilt, 0.0, 1.0) * deg
        vcs = ssum(vc)
        checks = [(jnp.abs(vcs[g] - target[g]) <= ELASTICITY * target[g])
                  .astype(jnp.float32) for g in range(G)]
        csum = functools.reduce(lambda p, q: p + q, checks)
        done = jnp.logical_or(done, csum >= G)
        return tuple(a_new), done

    a0 = tuple(jnp.float32(1.0) for _ in range(G))
    a, _ = jax.lax.fori_loop(0, NUM_ITER, body, (a0, jnp.array(False)))
    p_ref[...] = jnp.clip(bcast(list(a)) * xfilt * m, 0.0, 1.0)


def _loss_body(p_ref, d0_ref, d1_ref, s0_ref, s1_ref, b_ref, e_ref, l_ref):
    @pl.when(pl.program_id(0) == 0)
    def _():
        e_ref[...] = jnp.zeros((1, G), jnp.float32)
    v = p_ref[...] * ((d0_ref[...] + d1_ref[...])
                      - (s0_ref[...] + s1_ref[...]))
    b = b_ref[...]
    e_ref[...] += _row16([jnp.sum(jnp.where(b == g, v, 0.0))
                          for g in range(G)])

    @pl.when(pl.program_id(0) == pl.num_programs(0) - 1)
    def _():
        l_ref[...] = jnp.full((1, 1), 1.0, jnp.float32) * (
            jnp.sum(e_ref[...]) / G)


def _cspec(r, c):
    return pl.BlockSpec((r, c), lambda i: (i, 0))


def _kspec(shape):
    return pl.BlockSpec(shape, lambda i: (0, 0))


def _f32(shape):
    return jax.ShapeDtypeStruct(shape, jnp.float32)


NS = 16
NW = 2 * NS
CS = 10000
CV = 1000
SR = NP // NS
VR = NP // NS
ZR = 98
NZC = VR // ZR


def _mesh():
    return plsc.VectorSubcoreMesh(core_axis_name="c", subcore_axis_name="s")


_SC_PARAMS = pltpu.CompilerParams(use_tc_tiling_on_sc=False)


def _zero_vec(zb, nelem):
    def zf(j, _):
        zb[pl.ds(j * 16, 16)] = jnp.zeros((16,), jnp.float32)
        return 0
    lax.fori_loop(0, nelem // 16, zf, 0)


def _body_psum(e, vals, srci, dsti, out, acc, idx_v, val_v, zb, sem):
    c = lax.axis_index("c")
    s = lax.axis_index("s")
    _zero_vec(zb, SR)
    pltpu.sync_copy(zb, acc.at[pl.ds(s * SR, SR)])
    plsc.subcore_barrier()
    ew = e // NW
    t0 = (c * NS + s) * ew

    def ch(i, _):
        off = t0 + i * CS
        pltpu.sync_copy(srci.at[pl.ds(off, CS)], idx_v)
        pltpu.async_copy(vals.at[idx_v], val_v, sem).wait()
        pltpu.sync_copy(dsti.at[pl.ds(off, CS)], idx_v)
        pltpu.sync_copy(val_v, acc.at[idx_v], add=True)
        return 0
    lax.fori_loop(0, ew // CS, ch, 0)
    plsc.subcore_barrier()
    pltpu.sync_copy(acc.at[pl.ds(s * SR, SR)], zb)
    pltpu.sync_copy(zb, out.at[pl.ds(c * NP + s * SR, SR)])


def _body_pxdeg(e, vals, srci, dsti, outx, outd, accx, accd, idx_v, val_v,
                ones_v, zb, sem):
    c = lax.axis_index("c")
    s = lax.axis_index("s")
    _zero_vec(zb, SR)
    pltpu.sync_copy(zb, accx.at[pl.ds(s * SR, SR)])
    pltpu.sync_copy(zb, accd.at[pl.ds(s * SR, SR)])

    def of(j, _):
        ones_v[pl.ds(j * 16, 16)] = jnp.ones((16,), jnp.float32)
        return 0
    lax.fori_loop(0, CS // 16, of, 0)
    plsc.subcore_barrier()

    ew = e // NW
    t0 = (c * NS + s) * ew

    def ch(i, _):
        off = t0 + i * CS
        pltpu.sync_copy(srci.at[pl.ds(off, CS)], idx_v)
        pltpu.async_copy(vals.at[idx_v], val_v, sem).wait()
        pltpu.sync_copy(ones_v, accd.at[idx_v], add=True)
        pltpu.sync_copy(dsti.at[pl.ds(off, CS)], idx_v)
        pltpu.sync_copy(val_v, accx.at[idx_v], add=True)
        return 0
    lax.fori_loop(0, ew // CS, ch, 0)
    plsc.subcore_barrier()
    pltpu.sync_copy(accx.at[pl.ds(s * SR, SR)], zb)
    pltpu.sync_copy(zb, outx.at[pl.ds(c * NP + s * SR, SR)])
    pltpu.sync_copy(accd.at[pl.ds(s * SR, SR)], zb)
    pltpu.sync_copy(zb, outd.at[pl.ds(c * NP + s * SR, SR)])


def _body_gin(e, hcat, srci, dsti, outcat, acc, idx_v, didx_v, val_v, zb2,
              sem):
    c = lax.axis_index("c")
    s = lax.axis_index("s")

    def zf(j, _):
        zb2[j, :] = jnp.zeros((16,), jnp.float32)
        return 0
    lax.fori_loop(0, ZR, zf, 0)

    def zc(k, _):
        pltpu.sync_copy(zb2, acc.at[pl.ds(s * VR + k * ZR, ZR)])
        return 0
    lax.fori_loop(0, NZC, zc, 0)
    plsc.subcore_barrier()

    et = e // NS
    t0 = s * et

    def ch(i, _):
        off = t0 + i * CV
        pltpu.sync_copy(srci.at[pl.ds(off, CV)], idx_v)

        def rb(j, _):
            sl = pl.ds(j * 16, 16)
            idx_v[sl] = idx_v[sl] + c * NP
            return 0
        lax.fori_loop(0, CV // 16, rb, 0)
        pltpu.async_copy(hcat.at[idx_v], val_v, sem).wait()
        pltpu.sync_copy(dsti.at[pl.ds(off, CV)], didx_v)
        pltpu.sync_copy(val_v, acc.at[didx_v], add=True)
        return 0
    lax.fori_loop(0, et // CV, ch, 0)
    plsc.subcore_barrier()

    def oc(k, _):
        pltpu.sync_copy(acc.at[pl.ds(s * VR + k * ZR, ZR)], zb2)
        pltpu.sync_copy(zb2, outcat.at[pl.ds(c * NP + s * VR + k * ZR, ZR)])
        return 0
    lax.fori_loop(0, NZC, oc, 0)


def _sc_psum(vals, srci, dsti):
    e = srci.shape[0]
    return pl.kernel(
        functools.partial(_body_psum, e),
        out_type=jax.ShapeDtypeStruct((2 * NP,), jnp.float32),
        mesh=_mesh(),
        scratch_types=[
            pltpu.VMEM_SHARED((NP,), jnp.float32),
            pltpu.VMEM((CS,), jnp.int32),
            pltpu.VMEM((CS,), jnp.float32),
            pltpu.VMEM((SR,), jnp.float32),
            pltpu.SemaphoreType.DMA,
        ],
        compiler_params=_SC_PARAMS,
    )(vals, srci, dsti)


def _sc_pxdeg(x, srci, dsti):
    e = srci.shape[0]
    return pl.kernel(
        functools.partial(_body_pxdeg, e),
        out_type=[jax.ShapeDtypeStruct((2 * NP,), jnp.float32),
                  jax.ShapeDtypeStruct((2 * NP,), jnp.float32)],
        mesh=_mesh(),
        scratch_types=[
            pltpu.VMEM_SHARED((NP,), jnp.float32),
            pltpu.VMEM_SHARED((NP,), jnp.float32),
            pltpu.VMEM((CS,), jnp.int32),
            pltpu.VMEM((CS,), jnp.float32),
            pltpu.VMEM((CS,), jnp.float32),
            pltpu.VMEM((SR,), jnp.float32),
            pltpu.SemaphoreType.DMA,
        ],
        compiler_params=_SC_PARAMS,
    )(x, srci, dsti)


def _sc_edge_gin(hcat, srci, dsti):
    e = srci.shape[0]
    return pl.kernel(
        functools.partial(_body_gin, e),
        out_type=jax.ShapeDtypeStruct((2 * NP, 16), jnp.float32),
        mesh=_mesh(),
        scratch_types=[
            pltpu.VMEM_SHARED((NP, 16), jnp.float32),
            pltpu.VMEM((CV,), jnp.int32),
            pltpu.VMEM((CV,), jnp.int32),
            pltpu.VMEM((CV, 16), jnp.float32),
            pltpu.VMEM((ZR, 16), jnp.float32),
            pltpu.SemaphoreType.DMA,
        ],
        compiler_params=_SC_PARAMS,
    )(hcat, srci, dsti)


def _merge_thr(m, p):
    pp = p.reshape(2, NP // 128, 128)
    out = pl.pallas_call(
        _thr_body,
        grid=(NB,),
        in_specs=[_cspec(PR, 128), _cspec(PR, 128), _cspec(PR, 128)],
        out_specs=_cspec(PR, 128),
        out_shape=_f32((NP // 128, 128)),
    )(_plane(m), pp[0], pp[1])
    return out.reshape(NP)


def kernel(x, edge_index, batch, recfield_vol, total_vol, params):
    n = x.shape[0]
    row, col = edge_index[0], edge_index[1]
    pad = NP - n

    xp = jnp.pad(x, (0, pad))
    batchp = jnp.pad(batch, (0, pad), constant_values=G)

    m0 = jnp.pad((jnp.abs(x) > 0).astype(jnp.float32), (0, pad))
    m1 = _merge_thr(m0, _sc_psum(m0, row, col))
    px, pd = _sc_pxdeg(xp, row, col)
    pxr = px.reshape(2, NP)

    w1r = params['c1_W1'].reshape(1, H)
    gb = jnp.stack([params['c1_g'] * _BNS, params['c1_bt']])
    h0, h1 = pl.pallas_call(
        _layer1_body,
        grid=(NB,),
        in_specs=[_cspec(RB, 1), _cspec(RB, 1), _cspec(RB, 1), _cspec(RB, 1),
                  _kspec((1, H)), _kspec((1, H)), _kspec((H, H)),
                  _kspec((1, H)), _kspec((2, H))],
        out_specs=[_cspec(RB, 16), _cspec(RB, 16)],
        out_shape=[_f32((NP, 16)), _f32((NP, 16))],
    )(_col(xp), _col(pxr[0]), _col(pxr[1]), _col(m1), w1r,
      params['c1_b1'].reshape(1, H), params['c1_W2'],
      params['c1_b2'].reshape(1, H), gb)

    m_prev = m1
    for i in range(NUM_LAYERS - 1):
        m_new = _merge_thr(m_prev, _sc_psum(m_prev, row, col))
        acat = _sc_edge_gin(jnp.concatenate([h0, h1], axis=0), row, col)
        a0 = acat[:NP]
        a1 = acat[NP:]
        c = jnp.stack([params['cv%d_b1' % i], params['cv%d_b2' % i],
                       params['cv%d_g' % i] * _BNS, params['cv%d_bt' % i],
                       params['bn%d_g' % i] * _BNS, params['bn%d_bt' % i]])
        h0, h1 = pl.pallas_call(
            _gin_body,
            grid=(NB,),
            in_specs=[_cspec(RB, 16), _cspec(RB, 16), _cspec(RB, 16),
                      _cspec(RB, 16), _cspec(RB, 1),
                      _kspec((H, H)), _kspec((H, H)), _kspec((6, H))],
            out_specs=[_cspec(RB, 16), _cspec(RB, 16)],
            out_shape=[_f32((NP, 16)), _f32((NP, 16))],
        )(h0, h1, a0, a1, _col(m_new), params['cv%d_W1' % i],
          params['cv%d_W2' % i], c)
        m_prev = m_new

    m4 = m_prev

    c = jnp.zeros((4, H), jnp.float32)
    c = c.at[0].set(params['lin1_b'])
    c = c.at[1].set(params['bn2_g'] * _BNS)
    c = c.at[2].set(params['bn2_bt'])
    c = c.at[3, 0].set(params['lin2_b'][0])
    z = pl.pallas_call(
        _head_body,
        grid=(NB,),
        in_specs=[_cspec(RB, 16), _cspec(RB, 16), _cspec(RB, 1),
                  _kspec((H, H)), _kspec((4, H)), _kspec((H, 1))],
        out_specs=_cspec(RB, 1),
        out_shape=_f32((NP, 1)),
    )(h0, h1, _col(m4), params['lin1_W'], c, params['lin2_W'])
    z = z.reshape(NP)

    bmax, bmin = pl.pallas_call(
        _minmax_body,
        grid=(NB,),
        in_specs=[_cspec(PR, 128), _cspec(PR, 128)],
        out_specs=[_kspec((1, G)), _kspec((1, G))],
        out_shape=[_f32((1, G)), _f32((1, G))],
    )(_plane(z), _plane(batchp))

    kt = jax.random.fold_in(jax.random.key(1), 7)
    trand = jax.random.uniform(kt, (G,), jnp.float32).reshape(1, G)
    feas = (recfield_vol / total_vol).reshape(1, G)
    pdr = pd.reshape(2, NP // 128, 128)
    full = pl.BlockSpec((NP // 128, 128), lambda: (0, 0))
    probs = pl.pallas_call(
        _solver_body,
        grid=(),
        in_specs=[full, full, full, full, full, full,
                  pl.BlockSpec((1, G), lambda: (0, 0)),
                  pl.BlockSpec((1, G), lambda: (0, 0)),
                  pl.BlockSpec((1, G), lambda: (0, 0)),
                  pl.BlockSpec((1, G), lambda: (0, 0))],
        out_specs=full,
        out_shape=_f32((NP // 128, 128)),
    )(_plane(z), _plane(m4), _plane(xp), pdr[0], pdr[1], _plane(batchp),
      bmax, bmin, feas, trand)
    probs = probs.reshape(NP)

    ps = _sc_psum(probs, col, row).reshape(2, NP // 128, 128)
    e, loss = pl.pallas_call(
        _loss_body,
        grid=(NB,),
        in_specs=[_cspec(PR, 128), _cspec(PR, 128), _cspec(PR, 128),
                  _cspec(PR, 128), _cspec(PR, 128), _cspec(PR, 128)],
        out_specs=[_kspec((1, G)), _kspec((1, 1))],
        out_shape=[_f32((1, G)), _f32((1, 1))],
    )(_plane(probs), pdr[0], pdr[1], ps[0], ps[1], _plane(batchp))

    return probs[:n], loss[0, 0]
